# baseline (speedup 1.0000x reference)
import jax
import jax.numpy as jnp
from jax.experimental import pallas as pl
from jax.experimental.pallas import tpu as pltpu

_LC = 5.0
_LN = 0.5
_CS = 1.0 / 7.0

_CB = 1024
_W = 512


def _make_stage1(nsub):
    def _stage1(p_ref, t_ref, v_ref, o_ref, bc_ref):
        for k in range(nsub):
            sl = slice(k * _W, (k + 1) * _W)

            def pc(c):
                return p_ref[c, :, sl]

            def tc(c):
                return t_ref[c, :, sl]

            conf = tc(4)
            obj = conf == 1.0
            noobj = conf == 0.0
            nterm = _LN * (jnp.square(pc(4) - conf)
                           + jnp.square(pc(9) - tc(9)))

            p0, p1, p2, p3 = pc(0), pc(1), pc(2), pc(3)
            p5, p6, p7, p8 = pc(5), pc(6), pc(7), pc(8)
            a1x = p0 * _CS - p2
            a1y = p1 * _CS - p3
            b1x = a1x * _CS + p2
            b1y = a1y * _CS + p3
            a2x = p5 * _CS - p7
            a2y = p6 * _CS - p8
            b2x = a2x * _CS + p7
            b2y = a2y * _CS + p8
            t0, t1, t2, t3 = tc(0), tc(1), tc(2), tc(3)
            q0, q1, q2, q3 = t0 * t0, t1 * t1, t2 * t2, t3 * t3
            tax = q0 * _CS - q2
            tay = q1 * _CS - q3
            tbx = tax * _CS + q2
            tby = tay * _CS + q3
            area_t = (tbx - tax) * (tby - tay)

            def iou(ax, ay, bx, by):
                ltx = jnp.maximum(ax, tax)
                lty = jnp.maximum(ay, tay)
                rbx = jnp.minimum(bx, tbx)
                rby = jnp.minimum(by, tby)
                wx = jnp.maximum(rbx - ltx, 0.0)
                wy = jnp.maximum(rby - lty, 0.0)
                inter = wx * wy
                area_p = (bx - ax) * (by - ay)
                return inter / (area_p + area_t - inter)

            pick2 = iou(a2x, a2y, b2x, b2y) > iou(a1x, a1y, b1x, b1y)
            sx = jnp.where(pick2, p5, p0)
            sy = jnp.where(pick2, p6, p1)
            sw = jnp.where(pick2, p7, p2)
            sh = jnp.where(pick2, p8, p3)
            coord = (jnp.square(sx - t0) + jnp.square(sy - t1)
                     + jnp.square(sw - t2) + jnp.square(sh - t3))

            tcl = [tc(10 + c) for c in range(20)]
            m = tcl[0]
            for c in range(1, 20):
                m = jnp.maximum(m, tcl[c])
            idx = jnp.where(tcl[19] == m, 19, 20)
            for c in range(18, -1, -1):
                idx = jnp.where(tcl[c] == m, c, idx)
            selc = jnp.where(idx == 0, pc(10), 0.0)
            for c in range(1, 20):
                selc = selc + jnp.where(idx == c, pc(10 + c), 0.0)

            objterm = _LC * (coord + 2.0 * jnp.square(selc - 1.0))
            v = jnp.where(obj, objterm, jnp.where(noobj, nterm, 0.0))
            of = jnp.where(obj, 1.0, 0.0)
            v_ref[:, sl] = v
            o_ref[:, sl] = of
            bc_ref[k:k + 1] = jnp.sum(of, axis=0, keepdims=True
                                      ).reshape(1, 1, _W)

    return _stage1


def _make_stage2(nchunks):
    def _stage2(o_ref, v_ref, bc_ref, out_ref, utri, ex_ref):
        rr = jax.lax.broadcasted_iota(jnp.int32, (_W, _W), 0)
        cc = jax.lax.broadcasted_iota(jnp.int32, (_W, _W), 1)
        utri[...] = jnp.where(rr <= cc, 1.0, 0.0)

        bcs = bc_ref[...].reshape(nchunks, _W)
        prefb = jnp.dot(bcs, utri[...], preferred_element_type=jnp.float32)
        ct = prefb[:, _W - 1:_W]
        n = jnp.sum(ct, axis=0, keepdims=True)
        kcap = jnp.floor(n * 0.5)
        ctt = jnp.transpose(ct)
        rrc = jax.lax.broadcasted_iota(jnp.int32, (nchunks, nchunks), 0)
        ccc = jax.lax.broadcasted_iota(jnp.int32, (nchunks, nchunks), 1)
        cbase = jnp.sum(
            jnp.where(ccc < rrc, jnp.broadcast_to(ctt, (nchunks, nchunks)),
                      0.0),
            axis=1, keepdims=True)
        ex_ref[...] = (cbase + prefb - bcs).reshape(nchunks, 1, _W)

        rs = jax.lax.broadcasted_iota(jnp.int32, (49, 49), 0)
        cs = jax.lax.broadcasted_iota(jnp.int32, (49, 49), 1)
        ltri = jnp.where(rs >= cs, 1.0, 0.0)

        def body(j, acc):
            sl = pl.ds(pl.multiple_of(j * _W, _W), _W)
            ob = o_ref[:, sl]
            vb = v_ref[:, sl]
            cp = jnp.dot(ltri, ob, preferred_element_type=jnp.float32)
            rank = ex_ref[j] + cp
            drop = (ob == 1.0) & (rank > kcap)
            return acc + jnp.where(drop, 0.0, vb)

        acc = jax.lax.fori_loop(
            0, nchunks, body, jnp.zeros((49, _W), jnp.float32))
        out_ref[...] = jnp.sum(
            jnp.sum(acc, axis=0, keepdims=True), axis=1, keepdims=True)

    return _stage2


def kernel(predictions, targets):
    bsz = predictions.shape[0]
    nsub = _CB // _W
    steps = bsz // _CB
    nchunks = bsz // _W
    pt = predictions.transpose(2, 1, 0)
    tt = targets.transpose(2, 1, 0)

    v, o, bc = pl.pallas_call(
        _make_stage1(nsub),
        grid=(steps,),
        in_specs=[
            pl.BlockSpec((30, 49, _CB), lambda i: (0, 0, i)),
            pl.BlockSpec((30, 49, _CB), lambda i: (0, 0, i)),
        ],
        out_specs=[
            pl.BlockSpec((49, _CB), lambda i: (0, i)),
            pl.BlockSpec((49, _CB), lambda i: (0, i)),
            pl.BlockSpec((nsub, 1, _W), lambda i: (i, 0, 0)),
        ],
        out_shape=[
            jax.ShapeDtypeStruct((49, bsz), jnp.float32),
            jax.ShapeDtypeStruct((49, bsz), jnp.float32),
            jax.ShapeDtypeStruct((nchunks, 1, _W), jnp.float32),
        ],
        compiler_params=pltpu.CompilerParams(
            dimension_semantics=("arbitrary",),
            vmem_limit_bytes=56 * 1024 * 1024,
        ),
        name="yolo_loss_rows",
    )(pt, tt)

    loss = pl.pallas_call(
        _make_stage2(nchunks),
        out_shape=jax.ShapeDtypeStruct((1, 1), jnp.float32),
        scratch_shapes=[
            pltpu.VMEM((_W, _W), jnp.float32),
            pltpu.VMEM((nchunks, 1, _W), jnp.float32),
        ],
        name="yolo_loss_gate",
    )(o, v, bc)

    return loss[0, 0]
